# Initial kernel scaffold; baseline (speedup 1.0000x reference)
import jax
import jax.numpy as jnp
from jax.experimental import pallas as pl


def kernel(x, edge_index, batch, params):
    raise NotImplementedError("write your pallas kernel here")


# speedup vs baseline: 4.7531x; 4.7531x over previous
import functools

import jax
import jax.numpy as jnp
from jax import lax
from jax.experimental import pallas as pl
from jax.experimental.pallas import tpu as pltpu
from jax.experimental.pallas import tpu_sc as plsc

N = 10000
NP = 10240
E = 320000
B = 64
D = 128
MID = 64
FEAT = 3 * MID
HID = 128
OUT = 64
TAU = 0.5
W1C = 0.8

NW = 32
EPW = E // NW
CHUNK = 80
STEPS = EPW // CHUNK
RPT = NP // 16


def _build_seg_sum(dim):
    mesh = plsc.VectorSubcoreMesh(core_axis_name="c", subcore_axis_name="s")

    @functools.partial(
        pl.kernel,
        mesh=mesh,
        compiler_params=pltpu.CompilerParams(use_tc_tiling_on_sc=False),
        out_type=jax.ShapeDtypeStruct((2 * NP, dim), jnp.float32),
        scratch_types=[
            pltpu.VMEM((CHUNK,), jnp.int32),
            pltpu.VMEM((CHUNK,), jnp.int32),
            pltpu.VMEM((CHUNK, dim), jnp.float32),
            pltpu.VMEM_SHARED((NP, dim), jnp.float32),
            pltpu.SemaphoreType.DMA,
        ],
    )
    def seg_sum(h_hbm, src_hbm, dst_hbm, zeros_hbm, out_hbm,
                src_v, dst_v, rows_v, acc_sh, sem):
        cid = lax.axis_index("c")
        sid = lax.axis_index("s")
        wid = sid * 2 + cid
        pltpu.sync_copy(zeros_hbm.at[pl.ds(sid * RPT, RPT)],
                        acc_sh.at[pl.ds(sid * RPT, RPT)])
        plsc.subcore_barrier()

        base = wid * EPW

        def body(step, carry):
            off = base + step * CHUNK
            pltpu.sync_copy(src_hbm.at[pl.ds(off, CHUNK)], src_v)
            pltpu.sync_copy(dst_hbm.at[pl.ds(off, CHUNK)], dst_v)
            pltpu.async_copy(h_hbm.at[src_v], rows_v, sem).wait()
            pltpu.sync_copy(rows_v, acc_sh.at[dst_v], add=True)
            return carry

        lax.fori_loop(0, STEPS, body, 0)
        plsc.subcore_barrier()
        pltpu.sync_copy(acc_sh.at[pl.ds(sid * RPT, RPT)],
                        out_hbm.at[pl.ds(cid * NP + sid * RPT, RPT)])

    return seg_sum


def _edge_segment_sum(h_pad, src, dst, zeros):
    dim = h_pad.shape[1]
    out = _build_seg_sum(dim)(h_pad, src, dst, zeros)
    return out[:NP], out[NP:]


def _gin_mlp(h, p0, p1, epsp1, W1, b1, W2, b2):
    din = h.shape[1]
    blk = 1024

    def body(h_ref, p0_ref, p1_ref, eps_ref, w1_ref, b1_ref, w2_ref, b2_ref,
             o_ref):
        agg = h_ref[...] * eps_ref[0, 0] + p0_ref[...] + p1_ref[...]
        a1 = jnp.maximum(
            jnp.dot(agg, w1_ref[...], preferred_element_type=jnp.float32)
            + b1_ref[...], 0.0)
        o_ref[...] = jnp.maximum(
            jnp.dot(a1, w2_ref[...], preferred_element_type=jnp.float32)
            + b2_ref[...], 0.0)

    return pl.pallas_call(
        body,
        grid=(NP // blk,),
        in_specs=[
            pl.BlockSpec((blk, din), lambda i: (i, 0)),
            pl.BlockSpec((blk, din), lambda i: (i, 0)),
            pl.BlockSpec((blk, din), lambda i: (i, 0)),
            pl.BlockSpec(memory_space=pltpu.SMEM),
            pl.BlockSpec((din, MID), lambda i: (0, 0)),
            pl.BlockSpec((1, MID), lambda i: (0, 0)),
            pl.BlockSpec((MID, MID), lambda i: (0, 0)),
            pl.BlockSpec((1, MID), lambda i: (0, 0)),
        ],
        out_specs=pl.BlockSpec((blk, MID), lambda i: (i, 0)),
        out_shape=jax.ShapeDtypeStruct((NP, MID), jnp.float32),
    )(h, p0, p1, epsp1, W1, b1, W2, b2)


def _mlp_in(x, w1, b1, w2, b2):
    h = jnp.maximum(jnp.dot(x, w1, preferred_element_type=jnp.float32) + b1,
                    0.0)
    return jnp.dot(h, w2, preferred_element_type=jnp.float32) + b2


def _nt_xent_in(a, b):
    an = a / (jnp.sqrt(jnp.sum(a * a, axis=1, keepdims=True)) + 1e-8)
    bn = b / (jnp.sqrt(jnp.sum(b * b, axis=1, keepdims=True)) + 1e-8)
    sim = lax.dot_general(an, bn, (((1,), (1,)), ((), ())),
                          preferred_element_type=jnp.float32) / TAU
    mx = jnp.max(sim, axis=1, keepdims=True)
    logp = sim - mx - jnp.log(jnp.sum(jnp.exp(sim - mx), axis=1,
                                      keepdims=True))
    ii = lax.broadcasted_iota(jnp.int32, (B, B), 0)
    jj = lax.broadcasted_iota(jnp.int32, (B, B), 1)
    diag = jnp.sum(jnp.where(ii == jj, logp, 0.0), keepdims=True) / B
    return -diag


def _sortable_u32(s):
    s = jnp.where(s == 0.0, 0.0, s)
    u = lax.bitcast_convert_type(s, jnp.uint32)
    return jnp.where((u >> 31) == jnp.uint32(1), ~u,
                     u | jnp.uint32(0x80000000))


def _topk_gate(m, onehot, ksf):
    def s1_body(i, T):
        cand = T | (jnp.uint32(1) << (31 - i).astype(jnp.uint32))
        pred = jnp.where(m >= cand, onehot, 0.0)
        cnt = jnp.sum(pred, axis=1, keepdims=True)
        return jnp.where(cnt >= ksf, cand, T)

    T = lax.fori_loop(0, 32, s1_body, jnp.zeros((B, 1), jnp.uint32))
    gt = jnp.where(m > T, onehot, 0.0)
    tie = jnp.where(m == T, onehot, 0.0)
    r = ksf - jnp.sum(gt, axis=1, keepdims=True)
    key2 = jnp.uint32(NP) - lax.broadcasted_iota(jnp.uint32, (1, NP), 1)

    def s2_body(i, T2):
        cand = T2 | (jnp.uint32(1) << (13 - i).astype(jnp.uint32))
        cnt = jnp.sum(jnp.where(key2 >= cand, tie, 0.0), axis=1,
                      keepdims=True)
        return jnp.where(cnt >= r, cand, T2)

    T2 = lax.fori_loop(0, 14, s2_body, jnp.zeros((B, 1), jnp.uint32))
    sel_tie = jnp.where(key2 >= T2, tie, 0.0)
    return jnp.sum(gt, axis=0, keepdims=True) + jnp.sum(sel_tie, axis=0,
                                                        keepdims=True)


def _pool_heads(X, batch_row, w2T, pm, pl_, ps):
    def body(x_ref, b_ref, w_ref,
             mw1, mb1, mw2, mb2, lw1, lb1, lw2, lb2, sw1, sb1, sw2, sb2,
             o_ref):
        X_ = x_ref[...]
        brow = b_ref[...]
        gid = lax.broadcasted_iota(jnp.int32, (B, NP), 0)
        onehot = jnp.where(gid == brow, 1.0, 0.0)
        counts = jnp.sum(onehot, axis=1, keepdims=True)
        ksf = jnp.floor((3.0 * counts + 4.0) / 5.0)
        kdiv = jnp.maximum(ksf, 1.0)

        M1_P = lax.dot_general(onehot, X_, (((1,), (0,)), ((), ())),
                               preferred_element_type=jnp.float32)
        M1_con = _mlp_in(M1_P, mw1[...], mb1[...], mw2[...], mb2[...])

        wT = w_ref[...]
        wnorm = jnp.sqrt(jnp.sum(wT * wT, axis=1, keepdims=True))
        scores = lax.dot_general(wT, X_, (((1,), (1,)), ((), ())),
                                 preferred_element_type=jnp.float32)
        scores = scores / (wnorm + 1e-12)

        def branch(score, hw1, hb1, hw2, hb2):
            m = _sortable_u32(score)
            sel = _topk_gate(m, onehot, ksf)
            wgt = sel * jnp.tanh(score)
            num = lax.dot_general(onehot * wgt, X_, (((1,), (0,)), ((), ())),
                                  preferred_element_type=jnp.float32)
            readout = num / kdiv
            return _mlp_in(readout, hw1, hb1, hw2, hb2)

        L1_con = branch(scores[0:1], lw1[...], lb1[...], lw2[...], lb2[...])
        S1_con = branch(scores[1:2], sw1[...], sb1[...], sw2[...], sb2[...])
        yh = _nt_xent_in(M1_con, L1_con) + _nt_xent_in(M1_con, S1_con)
        o_ref[...] = W1C * yh

    args = [X, batch_row, w2T,
            pm['W1'], pm['b1'].reshape(1, HID), pm['W2'],
            pm['b2'].reshape(1, OUT),
            pl_['W1'], pl_['b1'].reshape(1, HID), pl_['W2'],
            pl_['b2'].reshape(1, OUT),
            ps['W1'], ps['b1'].reshape(1, HID), ps['W2'],
            ps['b2'].reshape(1, OUT)]
    return pl.pallas_call(
        body,
        out_shape=jax.ShapeDtypeStruct((1, 1), jnp.float32),
    )(*args)


def kernel(x, edge_index, batch, params):
    src = edge_index[0].astype(jnp.int32)
    dst = edge_index[1].astype(jnp.int32)
    h = jnp.pad(x, ((0, NP - N), (0, 0)))
    zeros_d = jnp.zeros((NP, D), jnp.float32)
    zeros_m = jnp.zeros((NP, MID), jnp.float32)
    xs = []
    for li, lp in enumerate(params['gin']):
        zeros = zeros_d if li == 0 else zeros_m
        p0, p1 = _edge_segment_sum(h, src, dst, zeros)
        epsp1 = (1.0 + lp['eps']).reshape(1, 1)
        h = _gin_mlp(h, p0, p1, epsp1, lp['W1'], lp['b1'].reshape(1, MID),
                     lp['W2'], lp['b2'].reshape(1, MID))
        xs.append(h)
    X = jnp.concatenate(xs, axis=1)
    batch_row = jnp.pad(batch.astype(jnp.int32), (0, NP - N),
                        constant_values=B).reshape(1, NP)
    w2T = jnp.stack([params['w_lp'], params['w_sp']], axis=0)
    out = _pool_heads(X, batch_row, w2T, params['ph_msg'], params['ph_loc'],
                      params['ph_sem'])
    return out.reshape(())


# speedup vs baseline: 9.7916x; 2.0600x over previous
import functools

import jax
import jax.numpy as jnp
from jax import lax
from jax.experimental import pallas as pl
from jax.experimental.pallas import tpu as pltpu
from jax.experimental.pallas import tpu_sc as plsc

N = 10000
NP = 10240
E = 320000
B = 64
D = 128
MID = 64
FEAT = 3 * MID
HID = 128
OUT = 64
TAU = 0.5
W1C = 0.8

NW = 32
EPW = E // NW
NBUF = 5
RPT = NP // 16


def _build_seg_sum(dim):
    CHUNK = 40 if dim > 64 else 80
    GROUPS = EPW // (CHUNK * NBUF)
    mesh = plsc.VectorSubcoreMesh(core_axis_name="c", subcore_axis_name="s")

    scratch = ([pltpu.VMEM((EPW,), jnp.int32)] +
               [pltpu.VMEM((CHUNK,), jnp.int32) for _ in range(NBUF)] +
               [pltpu.VMEM((CHUNK, dim), jnp.float32) for _ in range(NBUF)] +
               [pltpu.VMEM_SHARED((NP, dim), jnp.float32)] +
               [pltpu.SemaphoreType.DMA for _ in range(2 * NBUF + 1)])

    @functools.partial(
        pl.kernel,
        mesh=mesh,
        compiler_params=pltpu.CompilerParams(use_tc_tiling_on_sc=False),
        out_type=jax.ShapeDtypeStruct((2 * NP, dim), jnp.float32),
        scratch_types=scratch,
    )
    def seg_sum(h_hbm, src_hbm, dst_hbm, zeros_hbm, out_hbm, *refs):
        src_slab = refs[0]
        dst_v = refs[1:1 + NBUF]
        rows_v = refs[1 + NBUF:1 + 2 * NBUF]
        acc_sh = refs[1 + 2 * NBUF]
        semg = refs[2 + 2 * NBUF:2 + 3 * NBUF]
        sems = refs[2 + 3 * NBUF:2 + 4 * NBUF]
        semd = refs[2 + 4 * NBUF]

        cid = lax.axis_index("c")
        sid = lax.axis_index("s")
        wid = sid * 2 + cid
        base = wid * EPW
        pltpu.sync_copy(zeros_hbm.at[pl.ds(sid * RPT, RPT)],
                        acc_sh.at[pl.ds(sid * RPT, RPT)])
        pltpu.async_copy(src_hbm.at[pl.ds(base, EPW)], src_slab, semd).wait()
        plsc.subcore_barrier()

        def body(g, carry):
            goff = g * (CHUNK * NBUF)
            for b in range(NBUF):
                lo = goff + b * CHUNK
                pltpu.async_copy(dst_hbm.at[pl.ds(base + lo, CHUNK)],
                                 dst_v[b], semd)
                pltpu.async_copy(h_hbm.at[src_slab.at[pl.ds(lo, CHUNK)]],
                                 rows_v[b], semg[b])
            for b in range(NBUF):
                lo = goff + b * CHUNK
                pltpu.make_async_copy(dst_hbm.at[pl.ds(base + lo, CHUNK)],
                                      dst_v[b], semd).wait()
                pltpu.make_async_copy(h_hbm.at[src_slab.at[pl.ds(lo, CHUNK)]],
                                      rows_v[b], semg[b]).wait()
                pltpu.async_copy(rows_v[b], acc_sh.at[dst_v[b]], sems[b],
                                 add=True)
            for b in range(NBUF):
                pltpu.make_async_copy(rows_v[b], acc_sh.at[dst_v[b]],
                                      sems[b]).wait()
            return carry

        lax.fori_loop(0, GROUPS, body, 0)
        plsc.subcore_barrier()
        pltpu.sync_copy(acc_sh.at[pl.ds(sid * RPT, RPT)],
                        out_hbm.at[pl.ds(cid * NP + sid * RPT, RPT)])

    return seg_sum


def _edge_segment_sum(h_pad, src, dst, zeros):
    dim = h_pad.shape[1]
    out = _build_seg_sum(dim)(h_pad, src, dst, zeros)
    return out[:NP], out[NP:]


def _gin_mlp(h, p0, p1, epsp1, W1, b1, W2, b2):
    din = h.shape[1]
    blk = 1024

    def body(h_ref, p0_ref, p1_ref, eps_ref, w1_ref, b1_ref, w2_ref, b2_ref,
             o_ref):
        agg = h_ref[...] * eps_ref[0, 0] + p0_ref[...] + p1_ref[...]
        a1 = jnp.maximum(
            jnp.dot(agg, w1_ref[...], preferred_element_type=jnp.float32)
            + b1_ref[...], 0.0)
        o_ref[...] = jnp.maximum(
            jnp.dot(a1, w2_ref[...], preferred_element_type=jnp.float32)
            + b2_ref[...], 0.0)

    return pl.pallas_call(
        body,
        grid=(NP // blk,),
        in_specs=[
            pl.BlockSpec((blk, din), lambda i: (i, 0)),
            pl.BlockSpec((blk, din), lambda i: (i, 0)),
            pl.BlockSpec((blk, din), lambda i: (i, 0)),
            pl.BlockSpec(memory_space=pltpu.SMEM),
            pl.BlockSpec((din, MID), lambda i: (0, 0)),
            pl.BlockSpec((1, MID), lambda i: (0, 0)),
            pl.BlockSpec((MID, MID), lambda i: (0, 0)),
            pl.BlockSpec((1, MID), lambda i: (0, 0)),
        ],
        out_specs=pl.BlockSpec((blk, MID), lambda i: (i, 0)),
        out_shape=jax.ShapeDtypeStruct((NP, MID), jnp.float32),
    )(h, p0, p1, epsp1, W1, b1, W2, b2)


def _mlp_in(x, w1, b1, w2, b2):
    h = jnp.maximum(jnp.dot(x, w1, preferred_element_type=jnp.float32) + b1,
                    0.0)
    return jnp.dot(h, w2, preferred_element_type=jnp.float32) + b2


def _nt_xent_in(a, b):
    an = a / (jnp.sqrt(jnp.sum(a * a, axis=1, keepdims=True)) + 1e-8)
    bn = b / (jnp.sqrt(jnp.sum(b * b, axis=1, keepdims=True)) + 1e-8)
    sim = lax.dot_general(an, bn, (((1,), (1,)), ((), ())),
                          preferred_element_type=jnp.float32) / TAU
    mx = jnp.max(sim, axis=1, keepdims=True)
    logp = sim - mx - jnp.log(jnp.sum(jnp.exp(sim - mx), axis=1,
                                      keepdims=True))
    ii = lax.broadcasted_iota(jnp.int32, (B, B), 0)
    jj = lax.broadcasted_iota(jnp.int32, (B, B), 1)
    diag = jnp.sum(jnp.where(ii == jj, logp, 0.0), keepdims=True) / B
    return -diag


def _sortable_u32(s):
    s = jnp.where(s == 0.0, 0.0, s)
    u = lax.bitcast_convert_type(s, jnp.uint32)
    return jnp.where((u >> 31) == jnp.uint32(1), ~u,
                     u | jnp.uint32(0x80000000))


def _topk_gate(m, onehot, ksf):
    def s1_body(i, T):
        cand = T | (jnp.uint32(1) << (31 - i).astype(jnp.uint32))
        pred = jnp.where(m >= cand, onehot, 0.0)
        cnt = jnp.sum(pred, axis=1, keepdims=True)
        return jnp.where(cnt >= ksf, cand, T)

    T = lax.fori_loop(0, 32, s1_body, jnp.zeros((B, 1), jnp.uint32))
    gt = jnp.where(m > T, onehot, 0.0)
    tie = jnp.where(m == T, onehot, 0.0)
    r = ksf - jnp.sum(gt, axis=1, keepdims=True)
    key2 = jnp.uint32(NP) - lax.broadcasted_iota(jnp.uint32, (1, NP), 1)

    def s2_body(i, T2):
        cand = T2 | (jnp.uint32(1) << (13 - i).astype(jnp.uint32))
        cnt = jnp.sum(jnp.where(key2 >= cand, tie, 0.0), axis=1,
                      keepdims=True)
        return jnp.where(cnt >= r, cand, T2)

    T2 = lax.fori_loop(0, 14, s2_body, jnp.zeros((B, 1), jnp.uint32))
    sel_tie = jnp.where(key2 >= T2, tie, 0.0)
    return jnp.sum(gt, axis=0, keepdims=True) + jnp.sum(sel_tie, axis=0,
                                                        keepdims=True)


def _pool_heads(X, batch_row, w2T, pm, pl_, ps):
    def body(x_ref, b_ref, w_ref,
             mw1, mb1, mw2, mb2, lw1, lb1, lw2, lb2, sw1, sb1, sw2, sb2,
             o_ref):
        X_ = x_ref[...]
        brow = b_ref[...]
        gid = lax.broadcasted_iota(jnp.int32, (B, NP), 0)
        onehot = jnp.where(gid == brow, 1.0, 0.0)
        counts = jnp.sum(onehot, axis=1, keepdims=True)
        ksf = jnp.floor((3.0 * counts + 4.0) / 5.0)
        kdiv = jnp.maximum(ksf, 1.0)

        M1_P = lax.dot_general(onehot, X_, (((1,), (0,)), ((), ())),
                               preferred_element_type=jnp.float32)
        M1_con = _mlp_in(M1_P, mw1[...], mb1[...], mw2[...], mb2[...])

        wT = w_ref[...]
        wnorm = jnp.sqrt(jnp.sum(wT * wT, axis=1, keepdims=True))
        scores = lax.dot_general(wT, X_, (((1,), (1,)), ((), ())),
                                 preferred_element_type=jnp.float32)
        scores = scores / (wnorm + 1e-12)

        def branch(score, hw1, hb1, hw2, hb2):
            m = _sortable_u32(score)
            sel = _topk_gate(m, onehot, ksf)
            wgt = sel * jnp.tanh(score)
            num = lax.dot_general(onehot * wgt, X_, (((1,), (0,)), ((), ())),
                                  preferred_element_type=jnp.float32)
            readout = num / kdiv
            return _mlp_in(readout, hw1, hb1, hw2, hb2)

        L1_con = branch(scores[0:1], lw1[...], lb1[...], lw2[...], lb2[...])
        S1_con = branch(scores[1:2], sw1[...], sb1[...], sw2[...], sb2[...])
        yh = _nt_xent_in(M1_con, L1_con) + _nt_xent_in(M1_con, S1_con)
        o_ref[...] = W1C * yh

    args = [X, batch_row, w2T,
            pm['W1'], pm['b1'].reshape(1, HID), pm['W2'],
            pm['b2'].reshape(1, OUT),
            pl_['W1'], pl_['b1'].reshape(1, HID), pl_['W2'],
            pl_['b2'].reshape(1, OUT),
            ps['W1'], ps['b1'].reshape(1, HID), ps['W2'],
            ps['b2'].reshape(1, OUT)]
    return pl.pallas_call(
        body,
        out_shape=jax.ShapeDtypeStruct((1, 1), jnp.float32),
    )(*args)


def kernel(x, edge_index, batch, params):
    src = edge_index[0].astype(jnp.int32)
    dst = edge_index[1].astype(jnp.int32)
    h = jnp.pad(x, ((0, NP - N), (0, 0)))
    zeros_d = jnp.zeros((NP, D), jnp.float32)
    zeros_m = jnp.zeros((NP, MID), jnp.float32)
    xs = []
    for li, lp in enumerate(params['gin']):
        zeros = zeros_d if li == 0 else zeros_m
        p0, p1 = _edge_segment_sum(h, src, dst, zeros)
        epsp1 = (1.0 + lp['eps']).reshape(1, 1)
        h = _gin_mlp(h, p0, p1, epsp1, lp['W1'], lp['b1'].reshape(1, MID),
                     lp['W2'], lp['b2'].reshape(1, MID))
        xs.append(h)
    X = jnp.concatenate(xs, axis=1)
    batch_row = jnp.pad(batch.astype(jnp.int32), (0, NP - N),
                        constant_values=B).reshape(1, NP)
    w2T = jnp.stack([params['w_lp'], params['w_sp']], axis=0)
    out = _pool_heads(X, batch_row, w2T, params['ph_msg'], params['ph_loc'],
                      params['ph_sem'])
    return out.reshape(())


# speedup vs baseline: 11.0849x; 1.1321x over previous
import functools

import jax
import jax.numpy as jnp
from jax import lax
from jax.experimental import pallas as pl
from jax.experimental.pallas import tpu as pltpu
from jax.experimental.pallas import tpu_sc as plsc

N = 10000
NP = 10240
E = 320000
B = 64
D = 128
MID = 64
FEAT = 3 * MID
HID = 128
OUT = 64
TAU = 0.5
W1C = 0.8

NW = 32
EPW = E // NW
NBUF = 5
RPT = NP // 16


def _build_seg_sum(dim):
    CHUNK = 40 if dim > 64 else 80
    GROUPS = EPW // (CHUNK * NBUF)
    mesh = plsc.VectorSubcoreMesh(core_axis_name="c", subcore_axis_name="s")

    scratch = ([pltpu.VMEM((EPW,), jnp.int32)] +
               [pltpu.VMEM((CHUNK,), jnp.int32) for _ in range(NBUF)] +
               [pltpu.VMEM((CHUNK, dim), jnp.float32) for _ in range(NBUF)] +
               [pltpu.VMEM_SHARED((NP, dim), jnp.float32)] +
               [pltpu.SemaphoreType.DMA for _ in range(2 * NBUF + 1)])

    @functools.partial(
        pl.kernel,
        mesh=mesh,
        compiler_params=pltpu.CompilerParams(use_tc_tiling_on_sc=False),
        out_type=jax.ShapeDtypeStruct((2 * NP, dim), jnp.float32),
        scratch_types=scratch,
    )
    def seg_sum(h_hbm, src_hbm, dst_hbm, zeros_hbm, out_hbm, *refs):
        src_slab = refs[0]
        dst_v = refs[1:1 + NBUF]
        rows_v = refs[1 + NBUF:1 + 2 * NBUF]
        acc_sh = refs[1 + 2 * NBUF]
        semg = refs[2 + 2 * NBUF:2 + 3 * NBUF]
        sems = refs[2 + 3 * NBUF:2 + 4 * NBUF]
        semd = refs[2 + 4 * NBUF]

        cid = lax.axis_index("c")
        sid = lax.axis_index("s")
        wid = sid * 2 + cid
        base = wid * EPW
        pltpu.sync_copy(zeros_hbm.at[pl.ds(sid * RPT, RPT)],
                        acc_sh.at[pl.ds(sid * RPT, RPT)])
        pltpu.async_copy(src_hbm.at[pl.ds(base, EPW)], src_slab, semd).wait()
        plsc.subcore_barrier()

        def body(g, carry):
            goff = g * (CHUNK * NBUF)
            for b in range(NBUF):
                lo = goff + b * CHUNK
                pltpu.async_copy(dst_hbm.at[pl.ds(base + lo, CHUNK)],
                                 dst_v[b], semd)
                pltpu.async_copy(h_hbm.at[src_slab.at[pl.ds(lo, CHUNK)]],
                                 rows_v[b], semg[b])
            for b in range(NBUF):
                lo = goff + b * CHUNK
                pltpu.make_async_copy(dst_hbm.at[pl.ds(base + lo, CHUNK)],
                                      dst_v[b], semd).wait()
                pltpu.make_async_copy(h_hbm.at[src_slab.at[pl.ds(lo, CHUNK)]],
                                      rows_v[b], semg[b]).wait()
                pltpu.async_copy(rows_v[b], acc_sh.at[dst_v[b]], sems[b],
                                 add=True)
            for b in range(NBUF):
                pltpu.make_async_copy(rows_v[b], acc_sh.at[dst_v[b]],
                                      sems[b]).wait()
            return carry

        lax.fori_loop(0, GROUPS, body, 0)
        plsc.subcore_barrier()
        pltpu.sync_copy(acc_sh.at[pl.ds(sid * RPT, RPT)],
                        out_hbm.at[pl.ds(cid * NP + sid * RPT, RPT)])

    return seg_sum


def _edge_segment_sum(h_pad, src, dst, zeros):
    dim = h_pad.shape[1]
    out = _build_seg_sum(dim)(h_pad, src, dst, zeros)
    return out[:NP], out[NP:]


def _premul(x, W1):
    din = x.shape[1]
    blk = 2048

    def body(x_ref, w_ref, o_ref):
        o_ref[...] = jnp.dot(x_ref[...], w_ref[...],
                             preferred_element_type=jnp.float32)

    return pl.pallas_call(
        body,
        grid=(NP // blk,),
        in_specs=[
            pl.BlockSpec((blk, din), lambda i: (i, 0)),
            pl.BlockSpec((din, MID), lambda i: (0, 0)),
        ],
        out_specs=pl.BlockSpec((blk, MID), lambda i: (i, 0)),
        out_shape=jax.ShapeDtypeStruct((NP, MID), jnp.float32),
    )(x, W1)


def _gin_mlp(g, p0, p1, epsp1, b1, W2, b2, W1n):
    blk = 2048
    two = W1n is not None

    def body(g_ref, p0_ref, p1_ref, eps_ref, b1_ref, w2_ref, b2_ref,
             *refs):
        a1 = jnp.maximum(g_ref[...] * eps_ref[0, 0] + p0_ref[...]
                         + p1_ref[...] + b1_ref[...], 0.0)
        h = jnp.maximum(
            jnp.dot(a1, w2_ref[...], preferred_element_type=jnp.float32)
            + b2_ref[...], 0.0)
        if two:
            w1n_ref, h_ref, g_next_ref = refs
            h_ref[...] = h
            g_next_ref[...] = jnp.dot(h, w1n_ref[...],
                                      preferred_element_type=jnp.float32)
        else:
            (h_ref,) = refs
            h_ref[...] = h

    in_specs = [
        pl.BlockSpec((blk, MID), lambda i: (i, 0)),
        pl.BlockSpec((blk, MID), lambda i: (i, 0)),
        pl.BlockSpec((blk, MID), lambda i: (i, 0)),
        pl.BlockSpec(memory_space=pltpu.SMEM),
        pl.BlockSpec((1, MID), lambda i: (0, 0)),
        pl.BlockSpec((MID, MID), lambda i: (0, 0)),
        pl.BlockSpec((1, MID), lambda i: (0, 0)),
    ]
    args = [g, p0, p1, epsp1, b1, W2, b2]
    if two:
        in_specs.append(pl.BlockSpec((MID, MID), lambda i: (0, 0)))
        args.append(W1n)
        out_specs = [pl.BlockSpec((blk, MID), lambda i: (i, 0))] * 2
        out_shape = [jax.ShapeDtypeStruct((NP, MID), jnp.float32)] * 2
    else:
        out_specs = pl.BlockSpec((blk, MID), lambda i: (i, 0))
        out_shape = jax.ShapeDtypeStruct((NP, MID), jnp.float32)

    return pl.pallas_call(
        body,
        grid=(NP // blk,),
        in_specs=in_specs,
        out_specs=out_specs,
        out_shape=out_shape,
    )(*args)


def _mlp_in(x, w1, b1, w2, b2):
    h = jnp.maximum(jnp.dot(x, w1, preferred_element_type=jnp.float32) + b1,
                    0.0)
    return jnp.dot(h, w2, preferred_element_type=jnp.float32) + b2


def _nt_xent_in(a, b):
    an = a / (jnp.sqrt(jnp.sum(a * a, axis=1, keepdims=True)) + 1e-8)
    bn = b / (jnp.sqrt(jnp.sum(b * b, axis=1, keepdims=True)) + 1e-8)
    sim = lax.dot_general(an, bn, (((1,), (1,)), ((), ())),
                          preferred_element_type=jnp.float32) / TAU
    mx = jnp.max(sim, axis=1, keepdims=True)
    logp = sim - mx - jnp.log(jnp.sum(jnp.exp(sim - mx), axis=1,
                                      keepdims=True))
    ii = lax.broadcasted_iota(jnp.int32, (B, B), 0)
    jj = lax.broadcasted_iota(jnp.int32, (B, B), 1)
    diag = jnp.sum(jnp.where(ii == jj, logp, 0.0), keepdims=True) / B
    return -diag


def _sortable_u32(s):
    s = jnp.where(s == 0.0, 0.0, s)
    u = lax.bitcast_convert_type(s, jnp.uint32)
    return jnp.where((u >> 31) == jnp.uint32(1), ~u,
                     u | jnp.uint32(0x80000000))


def _topk_gate(m, onehot, ksf):
    def s1_body(i, T):
        cand = T | (jnp.uint32(1) << (31 - i).astype(jnp.uint32))
        pred = jnp.where(m >= cand, onehot, 0.0)
        cnt = jnp.sum(pred, axis=1, keepdims=True)
        return jnp.where(cnt >= ksf, cand, T)

    T = lax.fori_loop(0, 32, s1_body, jnp.zeros((B, 1), jnp.uint32))
    gt = jnp.where(m > T, onehot, 0.0)
    tie = jnp.where(m == T, onehot, 0.0)
    r = ksf - jnp.sum(gt, axis=1, keepdims=True)
    key2 = jnp.uint32(NP) - lax.broadcasted_iota(jnp.uint32, (1, NP), 1)

    def s2_body(i, T2):
        cand = T2 | (jnp.uint32(1) << (13 - i).astype(jnp.uint32))
        cnt = jnp.sum(jnp.where(key2 >= cand, tie, 0.0), axis=1,
                      keepdims=True)
        return jnp.where(cnt >= r, cand, T2)

    T2 = lax.fori_loop(0, 14, s2_body, jnp.zeros((B, 1), jnp.uint32))
    sel_tie = jnp.where(key2 >= T2, tie, 0.0)
    return jnp.sum(gt, axis=0, keepdims=True) + jnp.sum(sel_tie, axis=0,
                                                        keepdims=True)


def _pool_heads(X, batch_row, w2T, pm, pl_, ps):
    def body(x_ref, b_ref, w_ref,
             mw1, mb1, mw2, mb2, lw1, lb1, lw2, lb2, sw1, sb1, sw2, sb2,
             o_ref):
        X_ = x_ref[...]
        brow = b_ref[...]
        gid = lax.broadcasted_iota(jnp.int32, (B, NP), 0)
        onehot = jnp.where(gid == brow, 1.0, 0.0)
        counts = jnp.sum(onehot, axis=1, keepdims=True)
        ksf = jnp.floor((3.0 * counts + 4.0) / 5.0)
        kdiv = jnp.maximum(ksf, 1.0)

        M1_P = lax.dot_general(onehot, X_, (((1,), (0,)), ((), ())),
                               preferred_element_type=jnp.float32)
        M1_con = _mlp_in(M1_P, mw1[...], mb1[...], mw2[...], mb2[...])

        wT = w_ref[...]
        wnorm = jnp.sqrt(jnp.sum(wT * wT, axis=1, keepdims=True))
        scores = lax.dot_general(wT, X_, (((1,), (1,)), ((), ())),
                                 preferred_element_type=jnp.float32)
        scores = scores / (wnorm + 1e-12)

        def branch(score, hw1, hb1, hw2, hb2):
            m = _sortable_u32(score)
            sel = _topk_gate(m, onehot, ksf)
            wgt = sel * jnp.tanh(score)
            num = lax.dot_general(onehot * wgt, X_, (((1,), (0,)), ((), ())),
                                  preferred_element_type=jnp.float32)
            readout = num / kdiv
            return _mlp_in(readout, hw1, hb1, hw2, hb2)

        L1_con = branch(scores[0:1], lw1[...], lb1[...], lw2[...], lb2[...])
        S1_con = branch(scores[1:2], sw1[...], sb1[...], sw2[...], sb2[...])
        yh = _nt_xent_in(M1_con, L1_con) + _nt_xent_in(M1_con, S1_con)
        o_ref[...] = W1C * yh

    args = [X, batch_row, w2T,
            pm['W1'], pm['b1'].reshape(1, HID), pm['W2'],
            pm['b2'].reshape(1, OUT),
            pl_['W1'], pl_['b1'].reshape(1, HID), pl_['W2'],
            pl_['b2'].reshape(1, OUT),
            ps['W1'], ps['b1'].reshape(1, HID), ps['W2'],
            ps['b2'].reshape(1, OUT)]
    return pl.pallas_call(
        body,
        out_shape=jax.ShapeDtypeStruct((1, 1), jnp.float32),
    )(*args)


def kernel(x, edge_index, batch, params):
    src = edge_index[0].astype(jnp.int32)
    dst = edge_index[1].astype(jnp.int32)
    xpad = jnp.pad(x, ((0, NP - N), (0, 0)))
    zeros_m = jnp.zeros((NP, MID), jnp.float32)
    gin = params['gin']
    g = _premul(xpad, gin[0]['W1'])
    xs = []
    for li, lp in enumerate(gin):
        p0, p1 = _edge_segment_sum(g, src, dst, zeros_m)
        epsp1 = (1.0 + lp['eps']).reshape(1, 1)
        W1n = gin[li + 1]['W1'] if li + 1 < len(gin) else None
        res = _gin_mlp(g, p0, p1, epsp1, lp['b1'].reshape(1, MID),
                       lp['W2'], lp['b2'].reshape(1, MID), W1n)
        if W1n is not None:
            h, g = res
        else:
            h = res
        xs.append(h)
    X = jnp.concatenate(xs, axis=1)
    batch_row = jnp.pad(batch.astype(jnp.int32), (0, NP - N),
                        constant_values=B).reshape(1, NP)
    w2T = jnp.stack([params['w_lp'], params['w_sp']], axis=0)
    out = _pool_heads(X, batch_row, w2T, params['ph_msg'], params['ph_loc'],
                      params['ph_sem'])
    return out.reshape(())


# speedup vs baseline: 12.4152x; 1.1200x over previous
import functools

import jax
import jax.numpy as jnp
from jax import lax
from jax.experimental import pallas as pl
from jax.experimental.pallas import tpu as pltpu
from jax.experimental.pallas import tpu_sc as plsc

N = 10000
NP = 10240
E = 320000
B = 64
D = 128
MID = 64
FEAT = 3 * MID
HID = 128
OUT = 64
TAU = 0.5
W1C = 0.8

NW = 32
EPW = E // NW
NBUF = 5
RPT = NP // 16


def _build_seg_sum(dim):
    CHUNK = 40
    GROUPS = EPW // (CHUNK * NBUF)
    mesh = plsc.VectorSubcoreMesh(core_axis_name="c", subcore_axis_name="s")

    NS = 2 * NBUF
    scratch = ([pltpu.VMEM((EPW,), jnp.int32)] +
               [pltpu.VMEM((CHUNK,), jnp.int32) for _ in range(NS)] +
               [pltpu.VMEM((CHUNK, dim), jnp.float32) for _ in range(NS)] +
               [pltpu.VMEM_SHARED((NP, dim), jnp.float32)] +
               [pltpu.SemaphoreType.DMA for _ in range(2 * NS + 1)])

    @functools.partial(
        pl.kernel,
        mesh=mesh,
        compiler_params=pltpu.CompilerParams(use_tc_tiling_on_sc=False),
        out_type=jax.ShapeDtypeStruct((2 * NP, dim), jnp.float32),
        scratch_types=scratch,
    )
    def seg_sum(h_hbm, src_hbm, dst_hbm, zeros_hbm, out_hbm, *refs):
        src_slab = refs[0]
        dst_v = refs[1:1 + NS]
        rows_v = refs[1 + NS:1 + 2 * NS]
        acc_sh = refs[1 + 2 * NS]
        semg = refs[2 + 2 * NS:2 + 3 * NS]
        sems = refs[2 + 3 * NS:2 + 4 * NS]
        semd = refs[2 + 4 * NS]

        cid = lax.axis_index("c")
        sid = lax.axis_index("s")
        wid = sid * 2 + cid
        base = wid * EPW
        pltpu.sync_copy(zeros_hbm.at[pl.ds(sid * RPT, RPT)],
                        acc_sh.at[pl.ds(sid * RPT, RPT)])
        pltpu.async_copy(src_hbm.at[pl.ds(base, EPW)], src_slab, semd).wait()
        plsc.subcore_barrier()

        slotA = list(range(NBUF))
        slotB = list(range(NBUF, NS))

        def fire(goff, slots):
            for j, b in enumerate(slots):
                lo = goff + j * CHUNK
                pltpu.async_copy(dst_hbm.at[pl.ds(base + lo, CHUNK)],
                                 dst_v[b], semd)
                pltpu.async_copy(h_hbm.at[src_slab.at[pl.ds(lo, CHUNK)]],
                                 rows_v[b], semg[b])

        def scatter(goff, slots):
            for j, b in enumerate(slots):
                lo = goff + j * CHUNK
                pltpu.make_async_copy(dst_hbm.at[pl.ds(base + lo, CHUNK)],
                                      dst_v[b], semd).wait()
                pltpu.make_async_copy(h_hbm.at[src_slab.at[pl.ds(lo, CHUNK)]],
                                      rows_v[b], semg[b]).wait()
                pltpu.async_copy(rows_v[b], acc_sh.at[dst_v[b]], sems[b],
                                 add=True)

        def drain(slots):
            for b in slots:
                pltpu.make_async_copy(rows_v[b], acc_sh.at[dst_v[b]],
                                      sems[b]).wait()

        GW = CHUNK * NBUF
        fire(0, slotA)

        def body(j, carry):
            ga = 2 * j * GW
            gb = ga + GW
            fire(gb, slotB)
            scatter(ga, slotA)
            drain(slotA)

            @pl.when(j + 1 < GROUPS // 2)
            def _():
                fire(gb + GW, slotA)
            scatter(gb, slotB)
            drain(slotB)
            return carry

        lax.fori_loop(0, GROUPS // 2, body, 0)
        plsc.subcore_barrier()
        pltpu.sync_copy(acc_sh.at[pl.ds(sid * RPT, RPT)],
                        out_hbm.at[pl.ds(cid * NP + sid * RPT, RPT)])

    return seg_sum


def _edge_segment_sum(h_pad, src, dst, zeros):
    dim = h_pad.shape[1]
    out = _build_seg_sum(dim)(h_pad, src, dst, zeros)
    return out[:NP], out[NP:]


def _premul(x, W1):
    din = x.shape[1]
    blk = 2048

    def body(x_ref, w_ref, o_ref):
        o_ref[...] = jnp.dot(x_ref[...], w_ref[...],
                             preferred_element_type=jnp.float32)

    return pl.pallas_call(
        body,
        grid=(NP // blk,),
        in_specs=[
            pl.BlockSpec((blk, din), lambda i: (i, 0)),
            pl.BlockSpec((din, MID), lambda i: (0, 0)),
        ],
        out_specs=pl.BlockSpec((blk, MID), lambda i: (i, 0)),
        out_shape=jax.ShapeDtypeStruct((NP, MID), jnp.float32),
    )(x, W1)


def _gin_mlp(g, p0, p1, epsp1, b1, W2, b2, W1n):
    blk = 2048
    two = W1n is not None

    def body(g_ref, p0_ref, p1_ref, eps_ref, b1_ref, w2_ref, b2_ref,
             *refs):
        a1 = jnp.maximum(g_ref[...] * eps_ref[0, 0] + p0_ref[...]
                         + p1_ref[...] + b1_ref[...], 0.0)
        h = jnp.maximum(
            jnp.dot(a1, w2_ref[...], preferred_element_type=jnp.float32)
            + b2_ref[...], 0.0)
        if two:
            w1n_ref, h_ref, g_next_ref = refs
            h_ref[...] = h
            g_next_ref[...] = jnp.dot(h, w1n_ref[...],
                                      preferred_element_type=jnp.float32)
        else:
            (h_ref,) = refs
            h_ref[...] = h

    in_specs = [
        pl.BlockSpec((blk, MID), lambda i: (i, 0)),
        pl.BlockSpec((blk, MID), lambda i: (i, 0)),
        pl.BlockSpec((blk, MID), lambda i: (i, 0)),
        pl.BlockSpec(memory_space=pltpu.SMEM),
        pl.BlockSpec((1, MID), lambda i: (0, 0)),
        pl.BlockSpec((MID, MID), lambda i: (0, 0)),
        pl.BlockSpec((1, MID), lambda i: (0, 0)),
    ]
    args = [g, p0, p1, epsp1, b1, W2, b2]
    if two:
        in_specs.append(pl.BlockSpec((MID, MID), lambda i: (0, 0)))
        args.append(W1n)
        out_specs = [pl.BlockSpec((blk, MID), lambda i: (i, 0))] * 2
        out_shape = [jax.ShapeDtypeStruct((NP, MID), jnp.float32)] * 2
    else:
        out_specs = pl.BlockSpec((blk, MID), lambda i: (i, 0))
        out_shape = jax.ShapeDtypeStruct((NP, MID), jnp.float32)

    return pl.pallas_call(
        body,
        grid=(NP // blk,),
        in_specs=in_specs,
        out_specs=out_specs,
        out_shape=out_shape,
    )(*args)


def _mlp_in(x, w1, b1, w2, b2):
    h = jnp.maximum(jnp.dot(x, w1, preferred_element_type=jnp.float32) + b1,
                    0.0)
    return jnp.dot(h, w2, preferred_element_type=jnp.float32) + b2


def _nt_xent_in(a, b):
    an = a / (jnp.sqrt(jnp.sum(a * a, axis=1, keepdims=True)) + 1e-8)
    bn = b / (jnp.sqrt(jnp.sum(b * b, axis=1, keepdims=True)) + 1e-8)
    sim = lax.dot_general(an, bn, (((1,), (1,)), ((), ())),
                          preferred_element_type=jnp.float32) / TAU
    mx = jnp.max(sim, axis=1, keepdims=True)
    logp = sim - mx - jnp.log(jnp.sum(jnp.exp(sim - mx), axis=1,
                                      keepdims=True))
    ii = lax.broadcasted_iota(jnp.int32, (B, B), 0)
    jj = lax.broadcasted_iota(jnp.int32, (B, B), 1)
    diag = jnp.sum(jnp.where(ii == jj, logp, 0.0), keepdims=True) / B
    return -diag


def _sortable_u32(s):
    s = jnp.where(s == 0.0, 0.0, s)
    u = lax.bitcast_convert_type(s, jnp.uint32)
    return jnp.where((u >> 31) == jnp.uint32(1), ~u,
                     u | jnp.uint32(0x80000000))


def _topk_gate(m, onehot, ksf):
    def s1_body(i, T):
        cand = T | (jnp.uint32(1) << (31 - i).astype(jnp.uint32))
        pred = jnp.where(m >= cand, onehot, 0.0)
        cnt = jnp.sum(pred, axis=1, keepdims=True)
        return jnp.where(cnt >= ksf, cand, T)

    T = lax.fori_loop(0, 32, s1_body, jnp.zeros((B, 1), jnp.uint32))
    gt = jnp.where(m > T, onehot, 0.0)
    tie = jnp.where(m == T, onehot, 0.0)
    r = ksf - jnp.sum(gt, axis=1, keepdims=True)
    key2 = jnp.uint32(NP) - lax.broadcasted_iota(jnp.uint32, (1, NP), 1)

    def s2_body(i, T2):
        cand = T2 | (jnp.uint32(1) << (13 - i).astype(jnp.uint32))
        cnt = jnp.sum(jnp.where(key2 >= cand, tie, 0.0), axis=1,
                      keepdims=True)
        return jnp.where(cnt >= r, cand, T2)

    T2 = lax.fori_loop(0, 14, s2_body, jnp.zeros((B, 1), jnp.uint32))
    sel_tie = jnp.where(key2 >= T2, tie, 0.0)
    return jnp.sum(gt, axis=0, keepdims=True) + jnp.sum(sel_tie, axis=0,
                                                        keepdims=True)


def _pool_heads(X, batch_row, w2T, pm, pl_, ps):
    def body(x_ref, b_ref, w_ref,
             mw1, mb1, mw2, mb2, lw1, lb1, lw2, lb2, sw1, sb1, sw2, sb2,
             o_ref):
        X_ = x_ref[...]
        brow = b_ref[...]
        gid = lax.broadcasted_iota(jnp.int32, (B, NP), 0)
        onehot = jnp.where(gid == brow, 1.0, 0.0)
        counts = jnp.sum(onehot, axis=1, keepdims=True)
        ksf = jnp.floor((3.0 * counts + 4.0) / 5.0)
        kdiv = jnp.maximum(ksf, 1.0)

        M1_P = lax.dot_general(onehot, X_, (((1,), (0,)), ((), ())),
                               preferred_element_type=jnp.float32)
        M1_con = _mlp_in(M1_P, mw1[...], mb1[...], mw2[...], mb2[...])

        wT = w_ref[...]
        wnorm = jnp.sqrt(jnp.sum(wT * wT, axis=1, keepdims=True))
        scores = lax.dot_general(wT, X_, (((1,), (1,)), ((), ())),
                                 preferred_element_type=jnp.float32)
        scores = scores / (wnorm + 1e-12)

        def branch(score, hw1, hb1, hw2, hb2):
            m = _sortable_u32(score)
            sel = _topk_gate(m, onehot, ksf)
            wgt = sel * jnp.tanh(score)
            num = lax.dot_general(onehot * wgt, X_, (((1,), (0,)), ((), ())),
                                  preferred_element_type=jnp.float32)
            readout = num / kdiv
            return _mlp_in(readout, hw1, hb1, hw2, hb2)

        L1_con = branch(scores[0:1], lw1[...], lb1[...], lw2[...], lb2[...])
        S1_con = branch(scores[1:2], sw1[...], sb1[...], sw2[...], sb2[...])
        yh = _nt_xent_in(M1_con, L1_con) + _nt_xent_in(M1_con, S1_con)
        o_ref[...] = W1C * yh

    args = [X, batch_row, w2T,
            pm['W1'], pm['b1'].reshape(1, HID), pm['W2'],
            pm['b2'].reshape(1, OUT),
            pl_['W1'], pl_['b1'].reshape(1, HID), pl_['W2'],
            pl_['b2'].reshape(1, OUT),
            ps['W1'], ps['b1'].reshape(1, HID), ps['W2'],
            ps['b2'].reshape(1, OUT)]
    return pl.pallas_call(
        body,
        out_shape=jax.ShapeDtypeStruct((1, 1), jnp.float32),
    )(*args)


def kernel(x, edge_index, batch, params):
    src = edge_index[0].astype(jnp.int32)
    dst = edge_index[1].astype(jnp.int32)
    xpad = jnp.pad(x, ((0, NP - N), (0, 0)))
    zeros_m = jnp.zeros((NP, MID), jnp.float32)
    gin = params['gin']
    g = _premul(xpad, gin[0]['W1'])
    xs = []
    for li, lp in enumerate(gin):
        p0, p1 = _edge_segment_sum(g, src, dst, zeros_m)
        epsp1 = (1.0 + lp['eps']).reshape(1, 1)
        W1n = gin[li + 1]['W1'] if li + 1 < len(gin) else None
        res = _gin_mlp(g, p0, p1, epsp1, lp['b1'].reshape(1, MID),
                       lp['W2'], lp['b2'].reshape(1, MID), W1n)
        if W1n is not None:
            h, g = res
        else:
            h = res
        xs.append(h)
    X = jnp.concatenate(xs, axis=1)
    batch_row = jnp.pad(batch.astype(jnp.int32), (0, NP - N),
                        constant_values=B).reshape(1, NP)
    w2T = jnp.stack([params['w_lp'], params['w_sp']], axis=0)
    out = _pool_heads(X, batch_row, w2T, params['ph_msg'], params['ph_loc'],
                      params['ph_sem'])
    return out.reshape(())


# speedup vs baseline: 13.6621x; 1.1004x over previous
import functools

import jax
import jax.numpy as jnp
from jax import lax
from jax.experimental import pallas as pl
from jax.experimental.pallas import tpu as pltpu
from jax.experimental.pallas import tpu_sc as plsc

N = 10000
NP = 10240
E = 320000
B = 64
D = 128
MID = 64
FEAT = 3 * MID
HID = 128
OUT = 64
TAU = 0.5
W1C = 0.8

NW = 32
EPW = E // NW
NBUF = 5
RPT = NP // 16


def _build_seg_sum(dim):
    CHUNK = 80
    GROUPS = EPW // (CHUNK * NBUF)
    mesh = plsc.VectorSubcoreMesh(core_axis_name="c", subcore_axis_name="s")

    NS = 2 * NBUF
    scratch = ([pltpu.VMEM((EPW,), jnp.int32)] +
               [pltpu.VMEM((CHUNK,), jnp.int32) for _ in range(NS)] +
               [pltpu.VMEM((CHUNK, dim), jnp.float32) for _ in range(NS)] +
               [pltpu.VMEM_SHARED((NP, dim), jnp.float32)] +
               [pltpu.SemaphoreType.DMA for _ in range(2 * NS + 1)])

    @functools.partial(
        pl.kernel,
        mesh=mesh,
        compiler_params=pltpu.CompilerParams(use_tc_tiling_on_sc=False),
        out_type=jax.ShapeDtypeStruct((2 * NP, dim), jnp.float32),
        scratch_types=scratch,
    )
    def seg_sum(h_hbm, src_hbm, dst_hbm, zeros_hbm, out_hbm, *refs):
        src_slab = refs[0]
        dst_v = refs[1:1 + NS]
        rows_v = refs[1 + NS:1 + 2 * NS]
        acc_sh = refs[1 + 2 * NS]
        semg = refs[2 + 2 * NS:2 + 3 * NS]
        sems = refs[2 + 3 * NS:2 + 4 * NS]
        semd = refs[2 + 4 * NS]

        cid = lax.axis_index("c")
        sid = lax.axis_index("s")
        wid = sid * 2 + cid
        base = wid * EPW
        pltpu.sync_copy(zeros_hbm.at[pl.ds(sid * RPT, RPT)],
                        acc_sh.at[pl.ds(sid * RPT, RPT)])
        pltpu.async_copy(src_hbm.at[pl.ds(base, EPW)], src_slab, semd).wait()
        plsc.subcore_barrier()

        slotA = list(range(NBUF))
        slotB = list(range(NBUF, NS))

        def fire(goff, slots):
            for j, b in enumerate(slots):
                lo = goff + j * CHUNK
                pltpu.async_copy(dst_hbm.at[pl.ds(base + lo, CHUNK)],
                                 dst_v[b], semd)
                pltpu.async_copy(h_hbm.at[src_slab.at[pl.ds(lo, CHUNK)]],
                                 rows_v[b], semg[b])

        def scatter(goff, slots):
            for j, b in enumerate(slots):
                lo = goff + j * CHUNK
                pltpu.make_async_copy(dst_hbm.at[pl.ds(base + lo, CHUNK)],
                                      dst_v[b], semd).wait()
                pltpu.make_async_copy(h_hbm.at[src_slab.at[pl.ds(lo, CHUNK)]],
                                      rows_v[b], semg[b]).wait()
                pltpu.async_copy(rows_v[b], acc_sh.at[dst_v[b]], sems[b],
                                 add=True)

        def drain(slots):
            for b in slots:
                pltpu.make_async_copy(rows_v[b], acc_sh.at[dst_v[b]],
                                      sems[b]).wait()

        GW = CHUNK * NBUF
        fire(0, slotA)

        def body(j, carry):
            ga = 2 * j * GW
            gb = ga + GW
            fire(gb, slotB)
            scatter(ga, slotA)
            drain(slotA)
            fire(gb + GW, slotA)
            scatter(gb, slotB)
            drain(slotB)
            return carry

        lax.fori_loop(0, GROUPS // 2, body, 0)
        scatter((GROUPS - 1) * GW, slotA)
        drain(slotA)
        plsc.subcore_barrier()
        pltpu.sync_copy(acc_sh.at[pl.ds(sid * RPT, RPT)],
                        out_hbm.at[pl.ds(cid * NP + sid * RPT, RPT)])

    return seg_sum


def _edge_segment_sum(h_pad, src, dst, zeros):
    dim = h_pad.shape[1]
    return _build_seg_sum(dim)(h_pad, src, dst, zeros)


def _premul(x, W1):
    din = x.shape[1]
    blk = 2048

    def body(x_ref, w_ref, o_ref):
        o_ref[...] = jnp.dot(x_ref[...], w_ref[...],
                             preferred_element_type=jnp.float32)

    return pl.pallas_call(
        body,
        grid=(NP // blk,),
        in_specs=[
            pl.BlockSpec((blk, din), lambda i: (i, 0)),
            pl.BlockSpec((din, MID), lambda i: (0, 0)),
        ],
        out_specs=pl.BlockSpec((blk, MID), lambda i: (i, 0)),
        out_shape=jax.ShapeDtypeStruct((NP, MID), jnp.float32),
    )(x, W1)


def _gin_mlp(g, parts, epsp1, b1, W2, b2, W1n):
    blk = 2048
    two = W1n is not None

    def body(g_ref, p0_ref, p1_ref, eps_ref, b1_ref, w2_ref, b2_ref,
             *refs):
        a1 = jnp.maximum(g_ref[...] * eps_ref[0, 0] + p0_ref[...]
                         + p1_ref[...] + b1_ref[...], 0.0)
        h = jnp.maximum(
            jnp.dot(a1, w2_ref[...], preferred_element_type=jnp.float32)
            + b2_ref[...], 0.0)
        if two:
            w1n_ref, h_ref, g_next_ref = refs
            h_ref[...] = h
            g_next_ref[...] = jnp.dot(h, w1n_ref[...],
                                      preferred_element_type=jnp.float32)
        else:
            (h_ref,) = refs
            h_ref[...] = h

    nblk = NP // blk
    in_specs = [
        pl.BlockSpec((blk, MID), lambda i: (i, 0)),
        pl.BlockSpec((blk, MID), lambda i: (i, 0)),
        pl.BlockSpec((blk, MID), lambda i: (i + nblk, 0)),
        pl.BlockSpec(memory_space=pltpu.SMEM),
        pl.BlockSpec((1, MID), lambda i: (0, 0)),
        pl.BlockSpec((MID, MID), lambda i: (0, 0)),
        pl.BlockSpec((1, MID), lambda i: (0, 0)),
    ]
    args = [g, parts, parts, epsp1, b1, W2, b2]
    if two:
        in_specs.append(pl.BlockSpec((MID, MID), lambda i: (0, 0)))
        args.append(W1n)
        out_specs = [pl.BlockSpec((blk, MID), lambda i: (i, 0))] * 2
        out_shape = [jax.ShapeDtypeStruct((NP, MID), jnp.float32)] * 2
    else:
        out_specs = pl.BlockSpec((blk, MID), lambda i: (i, 0))
        out_shape = jax.ShapeDtypeStruct((NP, MID), jnp.float32)

    return pl.pallas_call(
        body,
        grid=(NP // blk,),
        in_specs=in_specs,
        out_specs=out_specs,
        out_shape=out_shape,
    )(*args)


def _mlp_in(x, w1, b1, w2, b2):
    h = jnp.maximum(jnp.dot(x, w1, preferred_element_type=jnp.float32) + b1,
                    0.0)
    return jnp.dot(h, w2, preferred_element_type=jnp.float32) + b2


def _nt_xent_in(a, b):
    an = a / (jnp.sqrt(jnp.sum(a * a, axis=1, keepdims=True)) + 1e-8)
    bn = b / (jnp.sqrt(jnp.sum(b * b, axis=1, keepdims=True)) + 1e-8)
    sim = lax.dot_general(an, bn, (((1,), (1,)), ((), ())),
                          preferred_element_type=jnp.float32) / TAU
    mx = jnp.max(sim, axis=1, keepdims=True)
    logp = sim - mx - jnp.log(jnp.sum(jnp.exp(sim - mx), axis=1,
                                      keepdims=True))
    ii = lax.broadcasted_iota(jnp.int32, (B, B), 0)
    jj = lax.broadcasted_iota(jnp.int32, (B, B), 1)
    diag = jnp.sum(jnp.where(ii == jj, logp, 0.0), keepdims=True) / B
    return -diag


def _sortable_u32(s):
    s = jnp.where(s == 0.0, 0.0, s)
    u = lax.bitcast_convert_type(s, jnp.uint32)
    return jnp.where((u >> 31) == jnp.uint32(1), ~u,
                     u | jnp.uint32(0x80000000))


def _topk_gate(m, onehot, ksf):
    def s1_body(i, T):
        cand = T | (jnp.uint32(1) << (31 - i).astype(jnp.uint32))
        pred = jnp.where(m >= cand, onehot, 0.0)
        cnt = jnp.sum(pred, axis=1, keepdims=True)
        return jnp.where(cnt >= ksf, cand, T)

    T = lax.fori_loop(0, 32, s1_body, jnp.zeros((B, 1), jnp.uint32))
    gt = jnp.where(m > T, onehot, 0.0)
    tie = jnp.where(m == T, onehot, 0.0)
    r = ksf - jnp.sum(gt, axis=1, keepdims=True)
    key2 = jnp.uint32(NP) - lax.broadcasted_iota(jnp.uint32, (1, NP), 1)

    def s2_body(i, T2):
        cand = T2 | (jnp.uint32(1) << (13 - i).astype(jnp.uint32))
        cnt = jnp.sum(jnp.where(key2 >= cand, tie, 0.0), axis=1,
                      keepdims=True)
        return jnp.where(cnt >= r, cand, T2)

    T2 = lax.fori_loop(0, 14, s2_body, jnp.zeros((B, 1), jnp.uint32))
    sel_tie = jnp.where(key2 >= T2, tie, 0.0)
    return jnp.sum(gt, axis=0, keepdims=True) + jnp.sum(sel_tie, axis=0,
                                                        keepdims=True)


def _pool_heads(X, batch_row, w2T, pm, pl_, ps):
    def body(x_ref, b_ref, w_ref,
             mw1, mb1, mw2, mb2, lw1, lb1, lw2, lb2, sw1, sb1, sw2, sb2,
             o_ref):
        X_ = x_ref[...]
        brow = b_ref[...]
        gid = lax.broadcasted_iota(jnp.int32, (B, NP), 0)
        onehot = jnp.where(gid == brow, 1.0, 0.0)
        counts = jnp.sum(onehot, axis=1, keepdims=True)
        ksf = jnp.floor((3.0 * counts + 4.0) / 5.0)
        kdiv = jnp.maximum(ksf, 1.0)

        M1_P = lax.dot_general(onehot, X_, (((1,), (0,)), ((), ())),
                               preferred_element_type=jnp.float32)
        M1_con = _mlp_in(M1_P, mw1[...], mb1[...], mw2[...], mb2[...])

        wT = w_ref[...]
        wnorm = jnp.sqrt(jnp.sum(wT * wT, axis=1, keepdims=True))
        scores = lax.dot_general(wT, X_, (((1,), (1,)), ((), ())),
                                 preferred_element_type=jnp.float32)
        scores = scores / (wnorm + 1e-12)

        def branch(score, hw1, hb1, hw2, hb2):
            m = _sortable_u32(score)
            sel = _topk_gate(m, onehot, ksf)
            wgt = sel * jnp.tanh(score)
            num = lax.dot_general(onehot * wgt, X_, (((1,), (0,)), ((), ())),
                                  preferred_element_type=jnp.float32)
            readout = num / kdiv
            return _mlp_in(readout, hw1, hb1, hw2, hb2)

        L1_con = branch(scores[0:1], lw1[...], lb1[...], lw2[...], lb2[...])
        S1_con = branch(scores[1:2], sw1[...], sb1[...], sw2[...], sb2[...])
        yh = _nt_xent_in(M1_con, L1_con) + _nt_xent_in(M1_con, S1_con)
        o_ref[...] = W1C * yh

    args = [X, batch_row, w2T,
            pm['W1'], pm['b1'].reshape(1, HID), pm['W2'],
            pm['b2'].reshape(1, OUT),
            pl_['W1'], pl_['b1'].reshape(1, HID), pl_['W2'],
            pl_['b2'].reshape(1, OUT),
            ps['W1'], ps['b1'].reshape(1, HID), ps['W2'],
            ps['b2'].reshape(1, OUT)]
    return pl.pallas_call(
        body,
        out_shape=jax.ShapeDtypeStruct((1, 1), jnp.float32),
    )(*args)


def kernel(x, edge_index, batch, params):
    src = edge_index[0].astype(jnp.int32)
    dst = edge_index[1].astype(jnp.int32)
    xpad = jnp.pad(x, ((0, NP - N), (0, 0)))
    zeros_m = jnp.zeros((NP, MID), jnp.float32)
    gin = params['gin']
    g = _premul(xpad, gin[0]['W1'])
    xs = []
    for li, lp in enumerate(gin):
        parts = _edge_segment_sum(g, src, dst, zeros_m)
        epsp1 = (1.0 + lp['eps']).reshape(1, 1)
        W1n = gin[li + 1]['W1'] if li + 1 < len(gin) else None
        res = _gin_mlp(g, parts, epsp1, lp['b1'].reshape(1, MID),
                       lp['W2'], lp['b2'].reshape(1, MID), W1n)
        if W1n is not None:
            h, g = res
        else:
            h = res
        xs.append(h)
    X = jnp.concatenate(xs, axis=1)
    batch_row = jnp.pad(batch.astype(jnp.int32), (0, NP - N),
                        constant_values=B).reshape(1, NP)
    w2T = jnp.stack([params['w_lp'], params['w_sp']], axis=0)
    out = _pool_heads(X, batch_row, w2T, params['ph_msg'], params['ph_loc'],
                      params['ph_sem'])
    return out.reshape(())


# speedup vs baseline: 13.7453x; 1.0061x over previous
import functools

import jax
import jax.numpy as jnp
from jax import lax
from jax.experimental import pallas as pl
from jax.experimental.pallas import tpu as pltpu
from jax.experimental.pallas import tpu_sc as plsc

N = 10000
NP = 10240
E = 320000
B = 64
D = 128
MID = 64
FEAT = 3 * MID
HID = 128
OUT = 64
TAU = 0.5
W1C = 0.8

NW = 32
EPW = E // NW
NBUF = 5
RPT = NP // 16


def _build_seg_sum(dim):
    CHUNK = 80
    STEPS = EPW // CHUNK
    GROUPS = EPW // (CHUNK * NBUF)
    mesh = plsc.VectorSubcoreMesh(core_axis_name="c", subcore_axis_name="s")

    NS = 2 * NBUF
    scratch = ([pltpu.VMEM((EPW,), jnp.int32),
                pltpu.VMEM((STEPS, CHUNK), jnp.int32)] +
               [pltpu.VMEM((CHUNK, dim), jnp.float32) for _ in range(NS)] +
               [pltpu.VMEM_SHARED((NP, dim), jnp.float32)] +
               [pltpu.SemaphoreType.DMA for _ in range(2 * NS + 1)])

    @functools.partial(
        pl.kernel,
        mesh=mesh,
        compiler_params=pltpu.CompilerParams(use_tc_tiling_on_sc=False),
        out_type=jax.ShapeDtypeStruct((2 * NP, dim), jnp.float32),
        scratch_types=scratch,
    )
    def seg_sum(h_hbm, src_hbm, dst3_hbm, zeros_hbm, out_hbm, *refs):
        src_slab = refs[0]
        dst_slab = refs[1]
        rows_v = refs[2:2 + NS]
        acc_sh = refs[2 + NS]
        semg = refs[3 + NS:3 + 2 * NS]
        sems = refs[3 + 2 * NS:3 + 3 * NS]
        semd = refs[3 + 3 * NS]

        cid = lax.axis_index("c")
        sid = lax.axis_index("s")
        wid = sid * 2 + cid
        base = wid * EPW
        pltpu.sync_copy(zeros_hbm.at[pl.ds(sid * RPT, RPT)],
                        acc_sh.at[pl.ds(sid * RPT, RPT)])
        pltpu.async_copy(src_hbm.at[pl.ds(base, EPW)], src_slab, semd)
        pltpu.async_copy(dst3_hbm.at[wid], dst_slab, semd)
        pltpu.make_async_copy(src_hbm.at[pl.ds(base, EPW)], src_slab,
                              semd).wait()
        pltpu.make_async_copy(dst3_hbm.at[wid], dst_slab, semd).wait()
        plsc.subcore_barrier()

        slotA = list(range(NBUF))
        slotB = list(range(NBUF, NS))

        def fire(gs, slots):
            for j, b in enumerate(slots):
                lo = (gs + j) * CHUNK
                pltpu.async_copy(h_hbm.at[src_slab.at[pl.ds(lo, CHUNK)]],
                                 rows_v[b], semg[b])

        def scatter(gs, slots):
            for j, b in enumerate(slots):
                lo = (gs + j) * CHUNK
                pltpu.make_async_copy(h_hbm.at[src_slab.at[pl.ds(lo, CHUNK)]],
                                      rows_v[b], semg[b]).wait()
                pltpu.async_copy(rows_v[b], acc_sh.at[dst_slab.at[gs + j]],
                                 sems[b], add=True)

        def drain(gs, slots):
            for j, b in enumerate(slots):
                pltpu.make_async_copy(rows_v[b],
                                      acc_sh.at[dst_slab.at[gs + j]],
                                      sems[b]).wait()

        fire(0, slotA)

        def body(j, carry):
            ga = 2 * j * NBUF
            gb = ga + NBUF
            fire(gb, slotB)
            scatter(ga, slotA)
            drain(ga, slotA)
            fire(gb + NBUF, slotA)
            scatter(gb, slotB)
            drain(gb, slotB)
            return carry

        lax.fori_loop(0, GROUPS // 2, body, 0)
        scatter((GROUPS - 1) * NBUF, slotA)
        drain((GROUPS - 1) * NBUF, slotA)
        plsc.subcore_barrier()
        pltpu.sync_copy(acc_sh.at[pl.ds(sid * RPT, RPT)],
                        out_hbm.at[pl.ds(cid * NP + sid * RPT, RPT)])

    return seg_sum


def _edge_segment_sum(h_pad, src, dst, zeros):
    dim = h_pad.shape[1]
    return _build_seg_sum(dim)(h_pad, src, dst, zeros)


def _premul(x, W1):
    din = x.shape[1]
    blk = 2048

    def body(x_ref, w_ref, o_ref):
        o_ref[...] = jnp.dot(x_ref[...], w_ref[...],
                             preferred_element_type=jnp.float32)

    return pl.pallas_call(
        body,
        grid=(NP // blk,),
        in_specs=[
            pl.BlockSpec((blk, din), lambda i: (i, 0)),
            pl.BlockSpec((din, MID), lambda i: (0, 0)),
        ],
        out_specs=pl.BlockSpec((blk, MID), lambda i: (i, 0)),
        out_shape=jax.ShapeDtypeStruct((NP, MID), jnp.float32),
    )(x, W1)


def _gin_mlp(g, parts, epsp1, b1, W2, b2, W1n):
    blk = 2048
    two = W1n is not None

    def body(g_ref, p0_ref, p1_ref, eps_ref, b1_ref, w2_ref, b2_ref,
             *refs):
        a1 = jnp.maximum(g_ref[...] * eps_ref[0, 0] + p0_ref[...]
                         + p1_ref[...] + b1_ref[...], 0.0)
        h = jnp.maximum(
            jnp.dot(a1, w2_ref[...], preferred_element_type=jnp.float32)
            + b2_ref[...], 0.0)
        if two:
            w1n_ref, h_ref, g_next_ref = refs
            h_ref[...] = h
            g_next_ref[...] = jnp.dot(h, w1n_ref[...],
                                      preferred_element_type=jnp.float32)
        else:
            (h_ref,) = refs
            h_ref[...] = h

    nblk = NP // blk
    in_specs = [
        pl.BlockSpec((blk, MID), lambda i: (i, 0)),
        pl.BlockSpec((blk, MID), lambda i: (i, 0)),
        pl.BlockSpec((blk, MID), lambda i: (i + nblk, 0)),
        pl.BlockSpec(memory_space=pltpu.SMEM),
        pl.BlockSpec((1, MID), lambda i: (0, 0)),
        pl.BlockSpec((MID, MID), lambda i: (0, 0)),
        pl.BlockSpec((1, MID), lambda i: (0, 0)),
    ]
    args = [g, parts, parts, epsp1, b1, W2, b2]
    if two:
        in_specs.append(pl.BlockSpec((MID, MID), lambda i: (0, 0)))
        args.append(W1n)
        out_specs = [pl.BlockSpec((blk, MID), lambda i: (i, 0))] * 2
        out_shape = [jax.ShapeDtypeStruct((NP, MID), jnp.float32)] * 2
    else:
        out_specs = pl.BlockSpec((blk, MID), lambda i: (i, 0))
        out_shape = jax.ShapeDtypeStruct((NP, MID), jnp.float32)

    return pl.pallas_call(
        body,
        grid=(NP // blk,),
        in_specs=in_specs,
        out_specs=out_specs,
        out_shape=out_shape,
    )(*args)


def _mlp_in(x, w1, b1, w2, b2):
    h = jnp.maximum(jnp.dot(x, w1, preferred_element_type=jnp.float32) + b1,
                    0.0)
    return jnp.dot(h, w2, preferred_element_type=jnp.float32) + b2


def _nt_xent_in(a, b):
    an = a / (jnp.sqrt(jnp.sum(a * a, axis=1, keepdims=True)) + 1e-8)
    bn = b / (jnp.sqrt(jnp.sum(b * b, axis=1, keepdims=True)) + 1e-8)
    sim = lax.dot_general(an, bn, (((1,), (1,)), ((), ())),
                          preferred_element_type=jnp.float32) / TAU
    mx = jnp.max(sim, axis=1, keepdims=True)
    logp = sim - mx - jnp.log(jnp.sum(jnp.exp(sim - mx), axis=1,
                                      keepdims=True))
    ii = lax.broadcasted_iota(jnp.int32, (B, B), 0)
    jj = lax.broadcasted_iota(jnp.int32, (B, B), 1)
    diag = jnp.sum(jnp.where(ii == jj, logp, 0.0), keepdims=True) / B
    return -diag


def _sortable_u32(s):
    s = jnp.where(s == 0.0, 0.0, s)
    u = lax.bitcast_convert_type(s, jnp.uint32)
    return jnp.where((u >> 31) == jnp.uint32(1), ~u,
                     u | jnp.uint32(0x80000000))


def _topk_gate(m, onehot, ksf):
    def s1_body(i, T):
        cand = T | (jnp.uint32(1) << (31 - i).astype(jnp.uint32))
        pred = jnp.where(m >= cand, onehot, 0.0)
        cnt = jnp.sum(pred, axis=1, keepdims=True)
        return jnp.where(cnt >= ksf, cand, T)

    T = lax.fori_loop(0, 32, s1_body, jnp.zeros((B, 1), jnp.uint32))
    gt = jnp.where(m > T, onehot, 0.0)
    tie = jnp.where(m == T, onehot, 0.0)
    r = ksf - jnp.sum(gt, axis=1, keepdims=True)
    key2 = jnp.uint32(NP) - lax.broadcasted_iota(jnp.uint32, (1, NP), 1)

    def s2_body(i, T2):
        cand = T2 | (jnp.uint32(1) << (13 - i).astype(jnp.uint32))
        cnt = jnp.sum(jnp.where(key2 >= cand, tie, 0.0), axis=1,
                      keepdims=True)
        return jnp.where(cnt >= r, cand, T2)

    T2 = lax.fori_loop(0, 14, s2_body, jnp.zeros((B, 1), jnp.uint32))
    sel_tie = jnp.where(key2 >= T2, tie, 0.0)
    return jnp.sum(gt, axis=0, keepdims=True) + jnp.sum(sel_tie, axis=0,
                                                        keepdims=True)


def _pool_heads(X, batch_row, w2T, pm, pl_, ps):
    def body(x_ref, b_ref, w_ref,
             mw1, mb1, mw2, mb2, lw1, lb1, lw2, lb2, sw1, sb1, sw2, sb2,
             o_ref):
        X_ = x_ref[...]
        brow = b_ref[...]
        gid = lax.broadcasted_iota(jnp.int32, (B, NP), 0)
        onehot = jnp.where(gid == brow, 1.0, 0.0)
        counts = jnp.sum(onehot, axis=1, keepdims=True)
        ksf = jnp.floor((3.0 * counts + 4.0) / 5.0)
        kdiv = jnp.maximum(ksf, 1.0)

        M1_P = lax.dot_general(onehot, X_, (((1,), (0,)), ((), ())),
                               preferred_element_type=jnp.float32)
        M1_con = _mlp_in(M1_P, mw1[...], mb1[...], mw2[...], mb2[...])

        wT = w_ref[...]
        wnorm = jnp.sqrt(jnp.sum(wT * wT, axis=1, keepdims=True))
        scores = lax.dot_general(wT, X_, (((1,), (1,)), ((), ())),
                                 preferred_element_type=jnp.float32)
        scores = scores / (wnorm + 1e-12)

        def branch(score, hw1, hb1, hw2, hb2):
            m = _sortable_u32(score)
            sel = _topk_gate(m, onehot, ksf)
            wgt = sel * jnp.tanh(score)
            num = lax.dot_general(onehot * wgt, X_, (((1,), (0,)), ((), ())),
                                  preferred_element_type=jnp.float32)
            readout = num / kdiv
            return _mlp_in(readout, hw1, hb1, hw2, hb2)

        L1_con = branch(scores[0:1], lw1[...], lb1[...], lw2[...], lb2[...])
        S1_con = branch(scores[1:2], sw1[...], sb1[...], sw2[...], sb2[...])
        yh = _nt_xent_in(M1_con, L1_con) + _nt_xent_in(M1_con, S1_con)
        o_ref[...] = W1C * yh

    args = [X, batch_row, w2T,
            pm['W1'], pm['b1'].reshape(1, HID), pm['W2'],
            pm['b2'].reshape(1, OUT),
            pl_['W1'], pl_['b1'].reshape(1, HID), pl_['W2'],
            pl_['b2'].reshape(1, OUT),
            ps['W1'], ps['b1'].reshape(1, HID), ps['W2'],
            ps['b2'].reshape(1, OUT)]
    return pl.pallas_call(
        body,
        out_shape=jax.ShapeDtypeStruct((1, 1), jnp.float32),
    )(*args)


def kernel(x, edge_index, batch, params):
    src = edge_index[0].astype(jnp.int32)
    dst = edge_index[1].astype(jnp.int32).reshape(NW, EPW // 80, 80)
    xpad = jnp.pad(x, ((0, NP - N), (0, 0)))
    zeros_m = jnp.zeros((NP, MID), jnp.float32)
    gin = params['gin']
    g = _premul(xpad, gin[0]['W1'])
    xs = []
    for li, lp in enumerate(gin):
        parts = _edge_segment_sum(g, src, dst, zeros_m)
        epsp1 = (1.0 + lp['eps']).reshape(1, 1)
        W1n = gin[li + 1]['W1'] if li + 1 < len(gin) else None
        res = _gin_mlp(g, parts, epsp1, lp['b1'].reshape(1, MID),
                       lp['W2'], lp['b2'].reshape(1, MID), W1n)
        if W1n is not None:
            h, g = res
        else:
            h = res
        xs.append(h)
    X = jnp.concatenate(xs, axis=1)
    batch_row = jnp.pad(batch.astype(jnp.int32), (0, NP - N),
                        constant_values=B).reshape(1, NP)
    w2T = jnp.stack([params['w_lp'], params['w_sp']], axis=0)
    out = _pool_heads(X, batch_row, w2T, params['ph_msg'], params['ph_loc'],
                      params['ph_sem'])
    return out.reshape(())


# speedup vs baseline: 14.3444x; 1.0436x over previous
import functools

import jax
import jax.numpy as jnp
from jax import lax
from jax.experimental import pallas as pl
from jax.experimental.pallas import tpu as pltpu
from jax.experimental.pallas import tpu_sc as plsc

N = 10000
NP = 10240
E = 320000
B = 64
D = 128
MID = 64
FEAT = 3 * MID
HID = 128
OUT = 64
TAU = 0.5
W1C = 0.8

NW = 32
EPW = E // NW
NBUF = 5
RPT = NP // 16


def _build_seg_sum(dim):
    CHUNK = 80
    STEPS = EPW // CHUNK
    GROUPS = EPW // (CHUNK * NBUF)
    mesh = plsc.VectorSubcoreMesh(core_axis_name="c", subcore_axis_name="s")

    NS = 2 * NBUF
    scratch = ([pltpu.VMEM((EPW,), jnp.int32),
                pltpu.VMEM((STEPS, CHUNK), jnp.int32)] +
               [pltpu.VMEM((CHUNK, dim), jnp.float32) for _ in range(NS)] +
               [pltpu.VMEM_SHARED((NP, dim), jnp.float32)] +
               [pltpu.SemaphoreType.DMA for _ in range(2 * NS + 1)])

    @functools.partial(
        pl.kernel,
        mesh=mesh,
        compiler_params=pltpu.CompilerParams(use_tc_tiling_on_sc=False),
        out_type=jax.ShapeDtypeStruct((2 * NP, dim), jnp.float32),
        scratch_types=scratch,
    )
    def seg_sum(h_hbm, src_hbm, dst3_hbm, zeros_hbm, out_hbm, *refs):
        src_slab = refs[0]
        dst_slab = refs[1]
        rows_v = refs[2:2 + NS]
        acc_sh = refs[2 + NS]
        semg = refs[3 + NS:3 + 2 * NS]
        sems = refs[3 + 2 * NS:3 + 3 * NS]
        semd = refs[3 + 3 * NS]

        cid = lax.axis_index("c")
        sid = lax.axis_index("s")
        wid = sid * 2 + cid
        base = wid * EPW
        pltpu.sync_copy(zeros_hbm.at[pl.ds(sid * RPT, RPT)],
                        acc_sh.at[pl.ds(sid * RPT, RPT)])
        pltpu.async_copy(src_hbm.at[pl.ds(base, EPW)], src_slab, semd)
        pltpu.async_copy(dst3_hbm.at[wid], dst_slab, semd)
        pltpu.make_async_copy(src_hbm.at[pl.ds(base, EPW)], src_slab,
                              semd).wait()
        pltpu.make_async_copy(dst3_hbm.at[wid], dst_slab, semd).wait()
        plsc.subcore_barrier()

        slotA = list(range(NBUF))
        slotB = list(range(NBUF, NS))

        def fire(gs, slots):
            for j, b in enumerate(slots):
                lo = (gs + j) * CHUNK
                pltpu.async_copy(h_hbm.at[src_slab.at[pl.ds(lo, CHUNK)]],
                                 rows_v[b], semg[b])

        def scatter(gs, slots):
            for j, b in enumerate(slots):
                lo = (gs + j) * CHUNK
                pltpu.make_async_copy(h_hbm.at[src_slab.at[pl.ds(lo, CHUNK)]],
                                      rows_v[b], semg[b]).wait()
                pltpu.async_copy(rows_v[b], acc_sh.at[dst_slab.at[gs + j]],
                                 sems[b], add=True)

        def drain(gs, slots):
            for j, b in enumerate(slots):
                pltpu.make_async_copy(rows_v[b],
                                      acc_sh.at[dst_slab.at[gs + j]],
                                      sems[b]).wait()

        fire(0, slotA)

        def body(j, carry):
            ga = 2 * j * NBUF
            gb = ga + NBUF
            fire(gb, slotB)
            scatter(ga, slotA)
            drain(ga, slotA)
            fire(gb + NBUF, slotA)
            scatter(gb, slotB)
            drain(gb, slotB)
            return carry

        lax.fori_loop(0, GROUPS // 2, body, 0)
        scatter((GROUPS - 1) * NBUF, slotA)
        drain((GROUPS - 1) * NBUF, slotA)
        plsc.subcore_barrier()
        pltpu.sync_copy(acc_sh.at[pl.ds(sid * RPT, RPT)],
                        out_hbm.at[pl.ds(cid * NP + sid * RPT, RPT)])

    return seg_sum


def _edge_segment_sum(h_pad, src, dst, zeros):
    dim = h_pad.shape[1]
    return _build_seg_sum(dim)(h_pad, src, dst, zeros)


def _premul(x, W1):
    din = x.shape[1]
    blk = 2048

    def body(x_ref, w_ref, o_ref):
        o_ref[...] = jnp.dot(x_ref[...], w_ref[...],
                             preferred_element_type=jnp.float32)

    return pl.pallas_call(
        body,
        grid=(NP // blk,),
        in_specs=[
            pl.BlockSpec((blk, din), lambda i: (i, 0)),
            pl.BlockSpec((din, MID), lambda i: (0, 0)),
        ],
        out_specs=pl.BlockSpec((blk, MID), lambda i: (i, 0)),
        out_shape=jax.ShapeDtypeStruct((NP, MID), jnp.float32),
    )(x, W1)


def _gin_mlp(g, parts, epsp1, b1, W2, b2, W1n):
    blk = 2048
    two = W1n is not None

    def body(g_ref, p0_ref, p1_ref, eps_ref, b1_ref, w2_ref, b2_ref,
             *refs):
        a1 = jnp.maximum(g_ref[...] * eps_ref[0, 0] + p0_ref[...]
                         + p1_ref[...] + b1_ref[...], 0.0)
        h = jnp.maximum(
            jnp.dot(a1, w2_ref[...], preferred_element_type=jnp.float32)
            + b2_ref[...], 0.0)
        if two:
            w1n_ref, h_ref, g_next_ref = refs
            h_ref[...] = h
            g_next_ref[...] = jnp.dot(h, w1n_ref[...],
                                      preferred_element_type=jnp.float32)
        else:
            (h_ref,) = refs
            h_ref[...] = h

    nblk = NP // blk
    in_specs = [
        pl.BlockSpec((blk, MID), lambda i: (i, 0)),
        pl.BlockSpec((blk, MID), lambda i: (i, 0)),
        pl.BlockSpec((blk, MID), lambda i: (i + nblk, 0)),
        pl.BlockSpec(memory_space=pltpu.SMEM),
        pl.BlockSpec((1, MID), lambda i: (0, 0)),
        pl.BlockSpec((MID, MID), lambda i: (0, 0)),
        pl.BlockSpec((1, MID), lambda i: (0, 0)),
    ]
    args = [g, parts, parts, epsp1, b1, W2, b2]
    if two:
        in_specs.append(pl.BlockSpec((MID, MID), lambda i: (0, 0)))
        args.append(W1n)
        out_specs = [pl.BlockSpec((blk, MID), lambda i: (i, 0))] * 2
        out_shape = [jax.ShapeDtypeStruct((NP, MID), jnp.float32)] * 2
    else:
        out_specs = pl.BlockSpec((blk, MID), lambda i: (i, 0))
        out_shape = jax.ShapeDtypeStruct((NP, MID), jnp.float32)

    return pl.pallas_call(
        body,
        grid=(NP // blk,),
        in_specs=in_specs,
        out_specs=out_specs,
        out_shape=out_shape,
    )(*args)


def _mlp_in(x, w1, b1, w2, b2):
    h = jnp.maximum(jnp.dot(x, w1, preferred_element_type=jnp.float32) + b1,
                    0.0)
    return jnp.dot(h, w2, preferred_element_type=jnp.float32) + b2


def _nt_xent_in(a, b):
    an = a / (jnp.sqrt(jnp.sum(a * a, axis=1, keepdims=True)) + 1e-8)
    bn = b / (jnp.sqrt(jnp.sum(b * b, axis=1, keepdims=True)) + 1e-8)
    sim = lax.dot_general(an, bn, (((1,), (1,)), ((), ())),
                          preferred_element_type=jnp.float32) / TAU
    mx = jnp.max(sim, axis=1, keepdims=True)
    logp = sim - mx - jnp.log(jnp.sum(jnp.exp(sim - mx), axis=1,
                                      keepdims=True))
    ii = lax.broadcasted_iota(jnp.int32, (B, B), 0)
    jj = lax.broadcasted_iota(jnp.int32, (B, B), 1)
    diag = jnp.sum(jnp.where(ii == jj, logp, 0.0), keepdims=True) / B
    return -diag


def _sortable_u32(s):
    s = jnp.where(s == 0.0, 0.0, s)
    u = lax.bitcast_convert_type(s, jnp.uint32)
    return jnp.where((u >> 31) == jnp.uint32(1), ~u,
                     u | jnp.uint32(0x80000000))


def _topk_gate(m, onehot, ksf):
    def s1_body(i, T):
        cand = T | (jnp.uint32(1) << (31 - i).astype(jnp.uint32))
        pred = jnp.where(m >= cand, onehot, 0.0)
        cnt = jnp.sum(pred, axis=1, keepdims=True)
        return jnp.where(cnt >= ksf, cand, T)

    T = lax.fori_loop(0, 32, s1_body, jnp.zeros((B, 1), jnp.uint32))
    gt = jnp.where(m > T, onehot, 0.0)
    tie = jnp.where(m == T, onehot, 0.0)
    r = ksf - jnp.sum(gt, axis=1, keepdims=True)
    key2 = jnp.uint32(NP) - lax.broadcasted_iota(jnp.uint32, (1, NP), 1)

    def s2_body(i, T2):
        cand = T2 | (jnp.uint32(1) << (13 - i).astype(jnp.uint32))
        cnt = jnp.sum(jnp.where(key2 >= cand, tie, 0.0), axis=1,
                      keepdims=True)
        return jnp.where(cnt >= r, cand, T2)

    T2 = lax.fori_loop(0, 14, s2_body, jnp.zeros((B, 1), jnp.uint32))
    sel_tie = jnp.where(key2 >= T2, tie, 0.0)
    return jnp.sum(gt, axis=0, keepdims=True) + jnp.sum(sel_tie, axis=0,
                                                        keepdims=True)


def _pool_heads(h1, h2, g3, parts3, eps3, l3b1, l3W2, l3b2,
                batch_row, w2T, pm, pl_, ps):
    def body(h1_ref, h2_ref, g3_ref, p3_ref, e3_ref, l3b1_ref, l3w2_ref,
             l3b2_ref, b_ref, w_ref,
             mw1, mb1, mw2, mb2, lw1, lb1, lw2, lb2, sw1, sb1, sw2, sb2,
             o_ref):
        a1 = jnp.maximum(g3_ref[...] * e3_ref[0, 0]
                         + p3_ref[pl.ds(0, NP), :] + p3_ref[pl.ds(NP, NP), :]
                         + l3b1_ref[...], 0.0)
        h3 = jnp.maximum(
            jnp.dot(a1, l3w2_ref[...], preferred_element_type=jnp.float32)
            + l3b2_ref[...], 0.0)
        hs = [h1_ref[...], h2_ref[...], h3]

        brow = b_ref[...]
        gid = lax.broadcasted_iota(jnp.int32, (B, NP), 0)
        onehot = jnp.where(gid == brow, 1.0, 0.0)
        counts = jnp.sum(onehot, axis=1, keepdims=True)
        ksf = jnp.floor((3.0 * counts + 4.0) / 5.0)
        kdiv = jnp.maximum(ksf, 1.0)

        def seg_mats(weight_row):
            ow = onehot if weight_row is None else onehot * weight_row
            return jnp.concatenate(
                [lax.dot_general(ow, h, (((1,), (0,)), ((), ())),
                                 preferred_element_type=jnp.float32)
                 for h in hs], axis=1)

        M1_P = seg_mats(None)
        M1_con = _mlp_in(M1_P, mw1[...], mb1[...], mw2[...], mb2[...])

        wT = w_ref[...]
        wnorm = jnp.sqrt(jnp.sum(wT * wT, axis=1, keepdims=True))
        scores = sum(
            lax.dot_general(wT[:, MID * l:MID * (l + 1)], hs[l],
                            (((1,), (1,)), ((), ())),
                            preferred_element_type=jnp.float32)
            for l in range(3))
        scores = scores / (wnorm + 1e-12)

        def branch(score, hw1, hb1, hw2, hb2):
            m = _sortable_u32(score)
            sel = _topk_gate(m, onehot, ksf)
            wgt = sel * jnp.tanh(score)
            readout = seg_mats(wgt) / kdiv
            return _mlp_in(readout, hw1, hb1, hw2, hb2)

        L1_con = branch(scores[0:1], lw1[...], lb1[...], lw2[...], lb2[...])
        S1_con = branch(scores[1:2], sw1[...], sb1[...], sw2[...], sb2[...])
        yh = _nt_xent_in(M1_con, L1_con) + _nt_xent_in(M1_con, S1_con)
        o_ref[...] = W1C * yh

    args = [h1, h2, g3, parts3, eps3, l3b1, l3W2, l3b2, batch_row, w2T,
            pm['W1'], pm['b1'].reshape(1, HID), pm['W2'],
            pm['b2'].reshape(1, OUT),
            pl_['W1'], pl_['b1'].reshape(1, HID), pl_['W2'],
            pl_['b2'].reshape(1, OUT),
            ps['W1'], ps['b1'].reshape(1, HID), ps['W2'],
            ps['b2'].reshape(1, OUT)]
    return pl.pallas_call(
        body,
        out_shape=jax.ShapeDtypeStruct((1, 1), jnp.float32),
    )(*args)


def kernel(x, edge_index, batch, params):
    src = edge_index[0].astype(jnp.int32)
    dst = edge_index[1].astype(jnp.int32).reshape(NW, EPW // 80, 80)
    xpad = jnp.pad(x, ((0, NP - N), (0, 0)))
    zeros_m = jnp.zeros((NP, MID), jnp.float32)
    gin = params['gin']
    g = _premul(xpad, gin[0]['W1'])
    hs = []
    for li, lp in enumerate(gin[:2]):
        parts = _edge_segment_sum(g, src, dst, zeros_m)
        epsp1 = (1.0 + lp['eps']).reshape(1, 1)
        h, g = _gin_mlp(g, parts, epsp1, lp['b1'].reshape(1, MID),
                        lp['W2'], lp['b2'].reshape(1, MID),
                        gin[li + 1]['W1'])
        hs.append(h)
    parts3 = _edge_segment_sum(g, src, dst, zeros_m)
    lp3 = gin[2]
    batch_row = jnp.pad(batch.astype(jnp.int32), (0, NP - N),
                        constant_values=B).reshape(1, NP)
    w2T = jnp.stack([params['w_lp'], params['w_sp']], axis=0)
    out = _pool_heads(hs[0], hs[1], g, parts3,
                      (1.0 + lp3['eps']).reshape(1, 1),
                      lp3['b1'].reshape(1, MID), lp3['W2'],
                      lp3['b2'].reshape(1, MID),
                      batch_row, w2T, params['ph_msg'], params['ph_loc'],
                      params['ph_sem'])
    return out.reshape(())
